# baseline (speedup 1.0000x reference)
import jax
import jax.numpy as jnp
from jax.experimental import pallas as pl
from jax.experimental.pallas import tpu as pltpu

NUM_LAYERS = 4
D = 2048
BATCH = 4096
TILE_B = 256


def _layer_kernel(x_ref, c0_ref, w_ref, b_ref, h_ref, c1_ref):
    xb = x_ref[...]
    u0 = jnp.dot(xb, w_ref[:, :D], preferred_element_type=jnp.float32)
    u1 = jnp.dot(xb, w_ref[:, D:2 * D], preferred_element_type=jnp.float32)
    f = jax.nn.sigmoid(u1 + b_ref[0, :])
    c1 = f * c0_ref[...] + (1.0 - f) * u0
    g = jnp.tanh(c1)
    u2 = jnp.dot(xb, w_ref[:, 2 * D:], preferred_element_type=jnp.float32)
    r = jax.nn.sigmoid(u2 + b_ref[1, :])
    h = r * g + (1.0 - r) * xb.astype(jnp.float32)
    h_ref[...] = h.astype(h_ref.dtype)
    c1_ref[...] = c1


def _layer(x_bf16, c0, w_bf16, b2, h_dtype):
    nb = BATCH // TILE_B
    return pl.pallas_call(
        _layer_kernel,
        grid=(nb,),
        in_specs=[
            pl.BlockSpec((TILE_B, D), lambda i: (i, 0)),
            pl.BlockSpec((TILE_B, D), lambda i: (i, 0)),
            pl.BlockSpec((D, 3 * D), lambda i: (0, 0)),
            pl.BlockSpec((2, D), lambda i: (0, 0)),
        ],
        out_specs=[
            pl.BlockSpec((TILE_B, D), lambda i: (i, 0)),
            pl.BlockSpec((TILE_B, D), lambda i: (i, 0)),
        ],
        out_shape=[
            jax.ShapeDtypeStruct((BATCH, D), h_dtype),
            jax.ShapeDtypeStruct((BATCH, D), jnp.float32),
        ],
        compiler_params=pltpu.CompilerParams(
            dimension_semantics=("arbitrary",),
        ),
    )(x_bf16, c0, w_bf16, b2)


def kernel(input, c_0, W0, b0, W1, b1, W2, b2, W3, b3):
    Ws = [W0, W1, W2, W3]
    bs = [b0, b1, b2, b3]
    h = input.astype(jnp.bfloat16)
    c1_list = []
    for i in range(NUM_LAYERS):
        h_dtype = jnp.float32 if i == NUM_LAYERS - 1 else jnp.bfloat16
        h, c1 = _layer(h, c_0[i], Ws[i].astype(jnp.bfloat16),
                       bs[i].reshape(2, D), h_dtype)
        c1_list.append(c1)
    return (h, jnp.stack(c1_list))
